# Initial kernel scaffold; baseline (speedup 1.0000x reference)
import jax
import jax.numpy as jnp
from jax.experimental import pallas as pl


def kernel(x, edge_index, W1, b1, W2, b2, W4, b4):
    raise NotImplementedError("write your pallas kernel here")


# speedup vs baseline: 16.1241x; 16.1241x over previous
import functools

import jax
import jax.numpy as jnp
from jax import lax
from jax.experimental import pallas as pl
from jax.experimental.pallas import tpu as pltpu
from jax.experimental.pallas import tpu_sc as plsc

N_NODES = 10000
NPAD = 10240
N_EDGES = 320000
NC = 2
NS = 16
NW = NC * NS
CHUNK = 80
PER_W = N_EDGES // NW
NCH = PER_W // CHUNK
PER_S = N_EDGES // NS
NCH1 = PER_S // CHUNK
ROWS_PER_SUB = NPAD // NS
ZR = 64

_MESH = plsc.VectorSubcoreMesh(
    core_axis_name="c", subcore_axis_name="s", num_cores=NC, num_subcores=NS
)


def _fill_zeros(ref, nrows, d):
    per_row = d // 16

    def body(i, _):
        r = i // per_row
        col = (i % per_row) * 16
        ref[r, pl.ds(col, 16)] = jnp.zeros((16,), jnp.float32)
        return 0

    lax.fori_loop(0, nrows * per_row, body, 0)


def _zero_acc(acc_sh, zbuf_v, s, d):
    _fill_zeros(zbuf_v, ZR, d)

    def zcopy(j, _):
        pltpu.sync_copy(zbuf_v, acc_sh.at[pl.ds(s * ROWS_PER_SUB + j * ZR, ZR)])
        return 0

    lax.fori_loop(0, ROWS_PER_SUB // ZR, zcopy, 0)


def _deg_body(dst3_hbm, out_hbm, idx_v, ones_v, zbuf_v, deg_sh):
    c = lax.axis_index("c")
    s = lax.axis_index("s")
    wid = s * NC + c

    def fill_ones(i, _):
        ones_v[pl.ds(i * 16, 16)] = jnp.ones((16,), jnp.float32)
        return 0

    lax.fori_loop(0, CHUNK // 16, fill_ones, 0)

    def fill_z(i, _):
        zbuf_v[pl.ds(i * 16, 16)] = jnp.zeros((16,), jnp.float32)
        return 0

    lax.fori_loop(0, ROWS_PER_SUB // 16, fill_z, 0)
    pltpu.sync_copy(zbuf_v, deg_sh.at[pl.ds(s * ROWS_PER_SUB, ROWS_PER_SUB)])
    plsc.subcore_barrier()

    pltpu.sync_copy(dst3_hbm.at[wid], idx_v)

    def body(k, _):
        pltpu.sync_copy(ones_v, deg_sh.at[idx_v.at[k]], add=True)
        return 0

    lax.fori_loop(0, NCH, body, 0)
    plsc.subcore_barrier()
    sl = pl.ds(s * ROWS_PER_SUB, ROWS_PER_SUB)
    pltpu.sync_copy(deg_sh.at[sl], out_hbm.at[c, sl])


@functools.partial(
    pl.kernel,
    out_type=jax.ShapeDtypeStruct((NC, NPAD), jnp.float32),
    mesh=_MESH,
    compiler_params=pltpu.CompilerParams(use_tc_tiling_on_sc=False),
    scratch_types=[
        pltpu.VMEM((NCH, CHUNK), jnp.int32),
        pltpu.VMEM((CHUNK,), jnp.float32),
        pltpu.VMEM((ROWS_PER_SUB,), jnp.float32),
        pltpu.VMEM_SHARED((NPAD,), jnp.float32),
    ],
    name="gcn_deg",
)
def _deg(dst3_hbm, out_hbm, idx_v, ones_v, zbuf_v, deg_sh):
    _deg_body(dst3_hbm, out_hbm, idx_v, ones_v, zbuf_v, deg_sh)


def _prop1_body(hsA, hsB, src16, dst16, out_hbm, idx_s, idx_d, rows_v,
                zbuf_v, acc_sh):
    c = lax.axis_index("c")
    s = lax.axis_index("s")
    _zero_acc(acc_sh, zbuf_v, s, 64)
    plsc.subcore_barrier()

    pltpu.sync_copy(src16.at[s], idx_s)
    pltpu.sync_copy(dst16.at[s], idx_d)

    def body(k, _):
        @pl.when(c == 0)
        def _():
            pltpu.sync_copy(hsA.at[idx_s.at[k]], rows_v)

        @pl.when(c == 1)
        def _():
            pltpu.sync_copy(hsB.at[idx_s.at[k]], rows_v)

        pltpu.sync_copy(rows_v, acc_sh.at[idx_d.at[k]], add=True)
        return 0

    lax.fori_loop(0, NCH1, body, 0)
    plsc.subcore_barrier()
    sl = pl.ds(s * ROWS_PER_SUB, ROWS_PER_SUB)
    pltpu.sync_copy(acc_sh.at[sl], out_hbm.at[c, sl])


@functools.partial(
    pl.kernel,
    out_type=jax.ShapeDtypeStruct((NC, NPAD, 64), jnp.float32),
    mesh=_MESH,
    compiler_params=pltpu.CompilerParams(use_tc_tiling_on_sc=False),
    scratch_types=[
        pltpu.VMEM((NCH1, CHUNK), jnp.int32),
        pltpu.VMEM((NCH1, CHUNK), jnp.int32),
        pltpu.VMEM((CHUNK, 64), jnp.float32),
        pltpu.VMEM((ZR, 64), jnp.float32),
        pltpu.VMEM_SHARED((NPAD, 64), jnp.float32),
    ],
    name="gcn_prop1",
)
def _prop1(hsA, hsB, src16, dst16, out_hbm, idx_s, idx_d, rows_v, zbuf_v,
           acc_sh):
    _prop1_body(hsA, hsB, src16, dst16, out_hbm, idx_s, idx_d, rows_v,
                zbuf_v, acc_sh)


def _prop_body(hs_hbm, src3_hbm, dst3_hbm, out_hbm, idx_s, idx_d, rows_v,
               zbuf_v, acc_sh, *, d):
    c = lax.axis_index("c")
    s = lax.axis_index("s")
    wid = s * NC + c
    _zero_acc(acc_sh, zbuf_v, s, d)
    plsc.subcore_barrier()

    pltpu.sync_copy(src3_hbm.at[wid], idx_s)
    pltpu.sync_copy(dst3_hbm.at[wid], idx_d)

    def body(k, _):
        pltpu.sync_copy(hs_hbm.at[idx_s.at[k]], rows_v)
        pltpu.sync_copy(rows_v, acc_sh.at[idx_d.at[k]], add=True)
        return 0

    lax.fori_loop(0, NCH, body, 0)
    plsc.subcore_barrier()
    sl = pl.ds(s * ROWS_PER_SUB, ROWS_PER_SUB)
    pltpu.sync_copy(acc_sh.at[sl], out_hbm.at[c, sl])


def _make_prop(d):
    @functools.partial(
        pl.kernel,
        out_type=jax.ShapeDtypeStruct((NC, NPAD, d), jnp.float32),
        mesh=_MESH,
        compiler_params=pltpu.CompilerParams(use_tc_tiling_on_sc=False),
        scratch_types=[
            pltpu.VMEM((NCH, CHUNK), jnp.int32),
            pltpu.VMEM((NCH, CHUNK), jnp.int32),
            pltpu.VMEM((CHUNK, d), jnp.float32),
            pltpu.VMEM((ZR, d), jnp.float32),
            pltpu.VMEM_SHARED((NPAD, d), jnp.float32),
        ],
        name=f"gcn_prop_{d}",
    )
    def prop(hs_hbm, src3_hbm, dst3_hbm, out_hbm, idx_s, idx_d, rows_v,
             zbuf_v, acc_sh):
        _prop_body(hs_hbm, src3_hbm, dst3_hbm, out_hbm, idx_s, idx_d,
                   rows_v, zbuf_v, acc_sh, d=d)

    return prop


_prop64 = _make_prop(64)
_prop16 = _make_prop(16)

BLK = 256
GRID = NPAD // BLK


def _m1_body(x_ref, w_ref, d0_ref, d1_ref, hsA_ref, hsB_ref, dinv_ref):
    z = jnp.dot(x_ref[...], w_ref[...], preferred_element_type=jnp.float32)
    dinv = lax.rsqrt(d0_ref[...] + d1_ref[...] + 1.0)
    hs = z * dinv
    hsA_ref[...] = hs[:, :64]
    hsB_ref[...] = hs[:, 64:]
    dinv_ref[...] = dinv


def _m1(x, w1, d0, d1):
    din, dout = w1.shape
    return pl.pallas_call(
        _m1_body,
        grid=(GRID,),
        in_specs=[
            pl.BlockSpec((BLK, din), lambda i: (i, 0)),
            pl.BlockSpec((din, dout), lambda i: (0, 0)),
            pl.BlockSpec((BLK, 1), lambda i: (i, 0)),
            pl.BlockSpec((BLK, 1), lambda i: (i, 0)),
        ],
        out_specs=[
            pl.BlockSpec((BLK, 64), lambda i: (i, 0)),
            pl.BlockSpec((BLK, 64), lambda i: (i, 0)),
            pl.BlockSpec((BLK, 1), lambda i: (i, 0)),
        ],
        out_shape=[
            jax.ShapeDtypeStruct((NPAD, 64), jnp.float32),
            jax.ShapeDtypeStruct((NPAD, 64), jnp.float32),
            jax.ShapeDtypeStruct((NPAD, 1), jnp.float32),
        ],
    )(x, w1, d0, d1)


def _m2_body(a0_ref, a1_ref, hsA_ref, hsB_ref, dinv_ref, b_ref, w_ref,
             out_ref):
    dinv = dinv_ref[...]
    preA = dinv * (a0_ref[...] + hsA_ref[...]) + b_ref[:, :64]
    preB = dinv * (a1_ref[...] + hsB_ref[...]) + b_ref[:, 64:]
    h = jnp.concatenate([jnp.maximum(preA, 0.0), jnp.maximum(preB, 0.0)],
                        axis=1)
    z = jnp.dot(h, w_ref[...], preferred_element_type=jnp.float32)
    out_ref[...] = z * dinv


def _m2(a0, a1, hsA, hsB, dinv, b, w):
    din, dout = w.shape
    return pl.pallas_call(
        _m2_body,
        grid=(GRID,),
        in_specs=[
            pl.BlockSpec((BLK, 64), lambda i: (i, 0)),
            pl.BlockSpec((BLK, 64), lambda i: (i, 0)),
            pl.BlockSpec((BLK, 64), lambda i: (i, 0)),
            pl.BlockSpec((BLK, 64), lambda i: (i, 0)),
            pl.BlockSpec((BLK, 1), lambda i: (i, 0)),
            pl.BlockSpec((1, din), lambda i: (0, 0)),
            pl.BlockSpec((din, dout), lambda i: (0, 0)),
        ],
        out_specs=pl.BlockSpec((BLK, dout), lambda i: (i, 0)),
        out_shape=jax.ShapeDtypeStruct((NPAD, dout), jnp.float32),
    )(a0, a1, hsA, hsB, dinv, b, w)


def _m3_body(a0_ref, a1_ref, hs_ref, dinv_ref, b_ref, w_ref, out_ref):
    dinv = dinv_ref[...]
    pre = dinv * (a0_ref[...] + a1_ref[...] + hs_ref[...]) + b_ref[...]
    h = jnp.maximum(pre, 0.0)
    z = jnp.dot(h, w_ref[...], preferred_element_type=jnp.float32)
    out_ref[...] = z * dinv


def _m3(a0, a1, hs, dinv, b, w):
    din, dout = w.shape
    return pl.pallas_call(
        _m3_body,
        grid=(GRID,),
        in_specs=[
            pl.BlockSpec((BLK, din), lambda i: (i, 0)),
            pl.BlockSpec((BLK, din), lambda i: (i, 0)),
            pl.BlockSpec((BLK, din), lambda i: (i, 0)),
            pl.BlockSpec((BLK, 1), lambda i: (i, 0)),
            pl.BlockSpec((1, din), lambda i: (0, 0)),
            pl.BlockSpec((din, dout), lambda i: (0, 0)),
        ],
        out_specs=pl.BlockSpec((BLK, dout), lambda i: (i, 0)),
        out_shape=jax.ShapeDtypeStruct((NPAD, dout), jnp.float32),
    )(a0, a1, hs, dinv, b, w)


def _m4_body(a0_ref, a1_ref, hs_ref, dinv_ref, b_ref, out_ref):
    pre = (dinv_ref[...] * (a0_ref[...] + a1_ref[...] + hs_ref[...])
           + b_ref[...])
    m = jnp.max(pre, axis=1, keepdims=True)
    l = pre - m
    out_ref[...] = l - jnp.log(jnp.sum(jnp.exp(l), axis=1, keepdims=True))


def _m4(a0, a1, hs, dinv, b):
    d = hs.shape[1]
    return pl.pallas_call(
        _m4_body,
        grid=(GRID,),
        in_specs=[
            pl.BlockSpec((BLK, d), lambda i: (i, 0)),
            pl.BlockSpec((BLK, d), lambda i: (i, 0)),
            pl.BlockSpec((BLK, d), lambda i: (i, 0)),
            pl.BlockSpec((BLK, 1), lambda i: (i, 0)),
            pl.BlockSpec((1, d), lambda i: (0, 0)),
        ],
        out_specs=pl.BlockSpec((BLK, d), lambda i: (i, 0)),
        out_shape=jax.ShapeDtypeStruct((NPAD, d), jnp.float32),
    )(a0, a1, hs, dinv, b)


@jax.jit
def kernel(x, edge_index, W1, b1, W2, b2, W4, b4):
    src = edge_index[0]
    dst = edge_index[1]
    src3 = src.reshape(NW, NCH, CHUNK)
    dst3 = dst.reshape(NW, NCH, CHUNK)
    src16 = src.reshape(NS, NCH1, CHUNK)
    dst16 = dst.reshape(NS, NCH1, CHUNK)
    x_pad = jnp.pad(x, ((0, NPAD - N_NODES), (0, 0)))

    deg_p = _deg(dst3)
    d0 = deg_p[0].reshape(NPAD, 1)
    d1 = deg_p[1].reshape(NPAD, 1)

    hsA, hsB, dinv = _m1(x_pad, W1, d0, d1)
    agg1 = _prop1(hsA, hsB, src16, dst16)
    hs2 = _m2(agg1[0], agg1[1], hsA, hsB, dinv, b1.reshape(1, -1), W2)
    agg2 = _prop64(hs2, src3, dst3)
    hs3 = _m3(agg2[0], agg2[1], hs2, dinv, b2.reshape(1, -1), W4)
    agg3 = _prop16(hs3, src3, dst3)
    out = _m4(agg3[0], agg3[1], hs3, dinv, b4.reshape(1, -1))
    return out[:N_NODES]


# speedup vs baseline: 24.4947x; 1.5191x over previous
import functools

import jax
import jax.numpy as jnp
from jax import lax
from jax.experimental import pallas as pl
from jax.experimental.pallas import tpu as pltpu
from jax.experimental.pallas import tpu_sc as plsc

N_NODES = 10000
NPAD = 10240
N_EDGES = 320000
NC = 2
NS = 16
NW = NC * NS
CHUNK = 400
PER_W = N_EDGES // NW
NCH = PER_W // CHUNK
PER_S = N_EDGES // NS
NCH1 = PER_S // CHUNK
ROWS_PER_SUB = NPAD // NS
ZR = 64

_MESH = plsc.VectorSubcoreMesh(
    core_axis_name="c", subcore_axis_name="s", num_cores=NC, num_subcores=NS
)


def _fill_zeros(ref, nrows, d):
    per_row = d // 16

    def body(i, _):
        r = i // per_row
        col = (i % per_row) * 16
        ref[r, pl.ds(col, 16)] = jnp.zeros((16,), jnp.float32)
        return 0

    lax.fori_loop(0, nrows * per_row, body, 0)


def _zero_acc(acc_sh, zbuf_v, s, d):
    _fill_zeros(zbuf_v, ZR, d)

    def zcopy(j, _):
        pltpu.sync_copy(zbuf_v, acc_sh.at[pl.ds(s * ROWS_PER_SUB + j * ZR, ZR)])
        return 0

    lax.fori_loop(0, ROWS_PER_SUB // ZR, zcopy, 0)


def _deg_body(dst3_hbm, out_hbm, idx_v, ones_v, zbuf_v, deg_sh):
    c = lax.axis_index("c")
    s = lax.axis_index("s")
    wid = s * NC + c

    def fill_ones(i, _):
        ones_v[pl.ds(i * 16, 16)] = jnp.ones((16,), jnp.float32)
        return 0

    lax.fori_loop(0, CHUNK // 16, fill_ones, 0)

    def fill_z(i, _):
        zbuf_v[pl.ds(i * 16, 16)] = jnp.zeros((16,), jnp.float32)
        return 0

    lax.fori_loop(0, ROWS_PER_SUB // 16, fill_z, 0)
    pltpu.sync_copy(zbuf_v, deg_sh.at[pl.ds(s * ROWS_PER_SUB, ROWS_PER_SUB)])
    plsc.subcore_barrier()

    pltpu.sync_copy(dst3_hbm.at[wid], idx_v)

    def body(k, _):
        pltpu.sync_copy(ones_v, deg_sh.at[idx_v.at[k]], add=True)
        return 0

    lax.fori_loop(0, NCH, body, 0)
    plsc.subcore_barrier()
    sl = pl.ds(s * ROWS_PER_SUB, ROWS_PER_SUB)
    pltpu.sync_copy(deg_sh.at[sl], out_hbm.at[c, sl])


@functools.partial(
    pl.kernel,
    out_type=jax.ShapeDtypeStruct((NC, NPAD), jnp.float32),
    mesh=_MESH,
    compiler_params=pltpu.CompilerParams(use_tc_tiling_on_sc=False),
    scratch_types=[
        pltpu.VMEM((NCH, CHUNK), jnp.int32),
        pltpu.VMEM((CHUNK,), jnp.float32),
        pltpu.VMEM((ROWS_PER_SUB,), jnp.float32),
        pltpu.VMEM_SHARED((NPAD,), jnp.float32),
    ],
    name="gcn_deg",
)
def _deg(dst3_hbm, out_hbm, idx_v, ones_v, zbuf_v, deg_sh):
    _deg_body(dst3_hbm, out_hbm, idx_v, ones_v, zbuf_v, deg_sh)


def _prop1_body(hsA, hsB, src16, dst16, out_hbm, idx_s, idx_d, rows_v,
                zbuf_v, acc_sh):
    c = lax.axis_index("c")
    s = lax.axis_index("s")
    _zero_acc(acc_sh, zbuf_v, s, 64)
    plsc.subcore_barrier()

    pltpu.sync_copy(src16.at[s], idx_s)
    pltpu.sync_copy(dst16.at[s], idx_d)

    def body(k, _):
        @pl.when(c == 0)
        def _():
            pltpu.sync_copy(hsA.at[idx_s.at[k]], rows_v)

        @pl.when(c == 1)
        def _():
            pltpu.sync_copy(hsB.at[idx_s.at[k]], rows_v)

        pltpu.sync_copy(rows_v, acc_sh.at[idx_d.at[k]], add=True)
        return 0

    lax.fori_loop(0, NCH1, body, 0)
    plsc.subcore_barrier()
    sl = pl.ds(s * ROWS_PER_SUB, ROWS_PER_SUB)
    pltpu.sync_copy(acc_sh.at[sl], out_hbm.at[c, sl])


@functools.partial(
    pl.kernel,
    out_type=jax.ShapeDtypeStruct((NC, NPAD, 64), jnp.float32),
    mesh=_MESH,
    compiler_params=pltpu.CompilerParams(use_tc_tiling_on_sc=False),
    scratch_types=[
        pltpu.VMEM((NCH1, CHUNK), jnp.int32),
        pltpu.VMEM((NCH1, CHUNK), jnp.int32),
        pltpu.VMEM((CHUNK, 64), jnp.float32),
        pltpu.VMEM((ZR, 64), jnp.float32),
        pltpu.VMEM_SHARED((NPAD, 64), jnp.float32),
    ],
    name="gcn_prop1",
)
def _prop1(hsA, hsB, src16, dst16, out_hbm, idx_s, idx_d, rows_v, zbuf_v,
           acc_sh):
    _prop1_body(hsA, hsB, src16, dst16, out_hbm, idx_s, idx_d, rows_v,
                zbuf_v, acc_sh)


def _prop_body(hs_hbm, src3_hbm, dst3_hbm, out_hbm, idx_s, idx_d, rows_v,
               zbuf_v, acc_sh, *, d):
    c = lax.axis_index("c")
    s = lax.axis_index("s")
    wid = s * NC + c
    _zero_acc(acc_sh, zbuf_v, s, d)
    plsc.subcore_barrier()

    pltpu.sync_copy(src3_hbm.at[wid], idx_s)
    pltpu.sync_copy(dst3_hbm.at[wid], idx_d)

    def body(k, _):
        pltpu.sync_copy(hs_hbm.at[idx_s.at[k]], rows_v)
        pltpu.sync_copy(rows_v, acc_sh.at[idx_d.at[k]], add=True)
        return 0

    lax.fori_loop(0, NCH, body, 0)
    plsc.subcore_barrier()
    sl = pl.ds(s * ROWS_PER_SUB, ROWS_PER_SUB)
    pltpu.sync_copy(acc_sh.at[sl], out_hbm.at[c, sl])


def _make_prop(d):
    @functools.partial(
        pl.kernel,
        out_type=jax.ShapeDtypeStruct((NC, NPAD, d), jnp.float32),
        mesh=_MESH,
        compiler_params=pltpu.CompilerParams(use_tc_tiling_on_sc=False),
        scratch_types=[
            pltpu.VMEM((NCH, CHUNK), jnp.int32),
            pltpu.VMEM((NCH, CHUNK), jnp.int32),
            pltpu.VMEM((CHUNK, d), jnp.float32),
            pltpu.VMEM((ZR, d), jnp.float32),
            pltpu.VMEM_SHARED((NPAD, d), jnp.float32),
        ],
        name=f"gcn_prop_{d}",
    )
    def prop(hs_hbm, src3_hbm, dst3_hbm, out_hbm, idx_s, idx_d, rows_v,
             zbuf_v, acc_sh):
        _prop_body(hs_hbm, src3_hbm, dst3_hbm, out_hbm, idx_s, idx_d,
                   rows_v, zbuf_v, acc_sh, d=d)

    return prop


_prop64 = _make_prop(64)
_prop16 = _make_prop(16)

BLK = 256
GRID = NPAD // BLK


def _m1_body(x_ref, w_ref, d0_ref, d1_ref, hsA_ref, hsB_ref, dinv_ref):
    z = jnp.dot(x_ref[...], w_ref[...], preferred_element_type=jnp.float32)
    dinv = lax.rsqrt(d0_ref[...] + d1_ref[...] + 1.0)
    hs = z * dinv
    hsA_ref[...] = hs[:, :64]
    hsB_ref[...] = hs[:, 64:]
    dinv_ref[...] = dinv


def _m1(x, w1, d0, d1):
    din, dout = w1.shape
    return pl.pallas_call(
        _m1_body,
        grid=(GRID,),
        in_specs=[
            pl.BlockSpec((BLK, din), lambda i: (i, 0)),
            pl.BlockSpec((din, dout), lambda i: (0, 0)),
            pl.BlockSpec((BLK, 1), lambda i: (i, 0)),
            pl.BlockSpec((BLK, 1), lambda i: (i, 0)),
        ],
        out_specs=[
            pl.BlockSpec((BLK, 64), lambda i: (i, 0)),
            pl.BlockSpec((BLK, 64), lambda i: (i, 0)),
            pl.BlockSpec((BLK, 1), lambda i: (i, 0)),
        ],
        out_shape=[
            jax.ShapeDtypeStruct((NPAD, 64), jnp.float32),
            jax.ShapeDtypeStruct((NPAD, 64), jnp.float32),
            jax.ShapeDtypeStruct((NPAD, 1), jnp.float32),
        ],
    )(x, w1, d0, d1)


def _m2_body(a0_ref, a1_ref, hsA_ref, hsB_ref, dinv_ref, b_ref, w_ref,
             out_ref):
    dinv = dinv_ref[...]
    preA = dinv * (a0_ref[...] + hsA_ref[...]) + b_ref[:, :64]
    preB = dinv * (a1_ref[...] + hsB_ref[...]) + b_ref[:, 64:]
    h = jnp.concatenate([jnp.maximum(preA, 0.0), jnp.maximum(preB, 0.0)],
                        axis=1)
    z = jnp.dot(h, w_ref[...], preferred_element_type=jnp.float32)
    out_ref[...] = z * dinv


def _m2(a0, a1, hsA, hsB, dinv, b, w):
    din, dout = w.shape
    return pl.pallas_call(
        _m2_body,
        grid=(GRID,),
        in_specs=[
            pl.BlockSpec((BLK, 64), lambda i: (i, 0)),
            pl.BlockSpec((BLK, 64), lambda i: (i, 0)),
            pl.BlockSpec((BLK, 64), lambda i: (i, 0)),
            pl.BlockSpec((BLK, 64), lambda i: (i, 0)),
            pl.BlockSpec((BLK, 1), lambda i: (i, 0)),
            pl.BlockSpec((1, din), lambda i: (0, 0)),
            pl.BlockSpec((din, dout), lambda i: (0, 0)),
        ],
        out_specs=pl.BlockSpec((BLK, dout), lambda i: (i, 0)),
        out_shape=jax.ShapeDtypeStruct((NPAD, dout), jnp.float32),
    )(a0, a1, hsA, hsB, dinv, b, w)


def _m3_body(a0_ref, a1_ref, hs_ref, dinv_ref, b_ref, w_ref, out_ref):
    dinv = dinv_ref[...]
    pre = dinv * (a0_ref[...] + a1_ref[...] + hs_ref[...]) + b_ref[...]
    h = jnp.maximum(pre, 0.0)
    z = jnp.dot(h, w_ref[...], preferred_element_type=jnp.float32)
    out_ref[...] = z * dinv


def _m3(a0, a1, hs, dinv, b, w):
    din, dout = w.shape
    return pl.pallas_call(
        _m3_body,
        grid=(GRID,),
        in_specs=[
            pl.BlockSpec((BLK, din), lambda i: (i, 0)),
            pl.BlockSpec((BLK, din), lambda i: (i, 0)),
            pl.BlockSpec((BLK, din), lambda i: (i, 0)),
            pl.BlockSpec((BLK, 1), lambda i: (i, 0)),
            pl.BlockSpec((1, din), lambda i: (0, 0)),
            pl.BlockSpec((din, dout), lambda i: (0, 0)),
        ],
        out_specs=pl.BlockSpec((BLK, dout), lambda i: (i, 0)),
        out_shape=jax.ShapeDtypeStruct((NPAD, dout), jnp.float32),
    )(a0, a1, hs, dinv, b, w)


def _m4_body(a0_ref, a1_ref, hs_ref, dinv_ref, b_ref, out_ref):
    pre = (dinv_ref[...] * (a0_ref[...] + a1_ref[...] + hs_ref[...])
           + b_ref[...])
    m = jnp.max(pre, axis=1, keepdims=True)
    l = pre - m
    out_ref[...] = l - jnp.log(jnp.sum(jnp.exp(l), axis=1, keepdims=True))


def _m4(a0, a1, hs, dinv, b):
    d = hs.shape[1]
    return pl.pallas_call(
        _m4_body,
        grid=(GRID,),
        in_specs=[
            pl.BlockSpec((BLK, d), lambda i: (i, 0)),
            pl.BlockSpec((BLK, d), lambda i: (i, 0)),
            pl.BlockSpec((BLK, d), lambda i: (i, 0)),
            pl.BlockSpec((BLK, 1), lambda i: (i, 0)),
            pl.BlockSpec((1, d), lambda i: (0, 0)),
        ],
        out_specs=pl.BlockSpec((BLK, d), lambda i: (i, 0)),
        out_shape=jax.ShapeDtypeStruct((NPAD, d), jnp.float32),
    )(a0, a1, hs, dinv, b)


@jax.jit
def kernel(x, edge_index, W1, b1, W2, b2, W4, b4):
    src = edge_index[0]
    dst = edge_index[1]
    src3 = src.reshape(NW, NCH, CHUNK)
    dst3 = dst.reshape(NW, NCH, CHUNK)
    src16 = src.reshape(NS, NCH1, CHUNK)
    dst16 = dst.reshape(NS, NCH1, CHUNK)
    x_pad = jnp.pad(x, ((0, NPAD - N_NODES), (0, 0)))

    deg_p = _deg(dst3)
    d0 = deg_p[0].reshape(NPAD, 1)
    d1 = deg_p[1].reshape(NPAD, 1)

    hsA, hsB, dinv = _m1(x_pad, W1, d0, d1)
    agg1 = _prop1(hsA, hsB, src16, dst16)
    hs2 = _m2(agg1[0], agg1[1], hsA, hsB, dinv, b1.reshape(1, -1), W2)
    agg2 = _prop64(hs2, src3, dst3)
    hs3 = _m3(agg2[0], agg2[1], hs2, dinv, b2.reshape(1, -1), W4)
    agg3 = _prop16(hs3, src3, dst3)
    out = _m4(agg3[0], agg3[1], hs3, dinv, b4.reshape(1, -1))
    return out[:N_NODES]


# speedup vs baseline: 28.1665x; 1.1499x over previous
import functools

import jax
import jax.numpy as jnp
from jax import lax
from jax.experimental import pallas as pl
from jax.experimental.pallas import tpu as pltpu
from jax.experimental.pallas import tpu_sc as plsc

N_NODES = 10000
NPAD = 10240
N_EDGES = 320000
NC = 2
NS = 16
NW = NC * NS
CHUNK = 400
PER_W = N_EDGES // NW
NCH = PER_W // CHUNK
PER_S = N_EDGES // NS
NCH1 = PER_S // CHUNK
ROWS_PER_SUB = NPAD // NS
ZR = 64

_MESH = plsc.VectorSubcoreMesh(
    core_axis_name="c", subcore_axis_name="s", num_cores=NC, num_subcores=NS
)


def _fill_zeros(ref, nrows, d):
    per_row = d // 16

    def body(i, _):
        r = i // per_row
        col = (i % per_row) * 16
        ref[r, pl.ds(col, 16)] = jnp.zeros((16,), jnp.float32)
        return 0

    lax.fori_loop(0, nrows * per_row, body, 0)


def _zero_acc(acc_sh, zbuf_v, s, d):
    _fill_zeros(zbuf_v, ZR, d)

    def zcopy(j, _):
        pltpu.sync_copy(zbuf_v, acc_sh.at[pl.ds(s * ROWS_PER_SUB + j * ZR, ZR)])
        return 0

    lax.fori_loop(0, ROWS_PER_SUB // ZR, zcopy, 0)


def _deg_body(dst3_hbm, out_hbm, idx_v, ones_v, zbuf_v, deg_sh):
    c = lax.axis_index("c")
    s = lax.axis_index("s")
    wid = s * NC + c

    def fill_ones(i, _):
        ones_v[pl.ds(i * 16, 16)] = jnp.ones((16,), jnp.float32)
        return 0

    lax.fori_loop(0, CHUNK // 16, fill_ones, 0)

    def fill_z(i, _):
        zbuf_v[pl.ds(i * 16, 16)] = jnp.zeros((16,), jnp.float32)
        return 0

    lax.fori_loop(0, ROWS_PER_SUB // 16, fill_z, 0)
    pltpu.sync_copy(zbuf_v, deg_sh.at[pl.ds(s * ROWS_PER_SUB, ROWS_PER_SUB)])
    plsc.subcore_barrier()

    pltpu.sync_copy(dst3_hbm.at[wid], idx_v)

    def body(k, _):
        pltpu.sync_copy(ones_v, deg_sh.at[idx_v.at[k]], add=True)
        return 0

    lax.fori_loop(0, NCH, body, 0)
    plsc.subcore_barrier()
    sl = pl.ds(s * ROWS_PER_SUB, ROWS_PER_SUB)
    pltpu.sync_copy(deg_sh.at[sl], out_hbm.at[c, sl])


@functools.partial(
    pl.kernel,
    out_type=jax.ShapeDtypeStruct((NC, NPAD), jnp.float32),
    mesh=_MESH,
    compiler_params=pltpu.CompilerParams(use_tc_tiling_on_sc=False),
    scratch_types=[
        pltpu.VMEM((NCH, CHUNK), jnp.int32),
        pltpu.VMEM((CHUNK,), jnp.float32),
        pltpu.VMEM((ROWS_PER_SUB,), jnp.float32),
        pltpu.VMEM_SHARED((NPAD,), jnp.float32),
    ],
    name="gcn_deg",
)
def _deg(dst3_hbm, out_hbm, idx_v, ones_v, zbuf_v, deg_sh):
    _deg_body(dst3_hbm, out_hbm, idx_v, ones_v, zbuf_v, deg_sh)


def _prop1_body(hsA, hsB, src16, dst16, out_hbm, idx_s, idx_d, rows_v,
                zbuf_v, acc_sh):
    c = lax.axis_index("c")
    s = lax.axis_index("s")
    _zero_acc(acc_sh, zbuf_v, s, 64)
    plsc.subcore_barrier()

    pltpu.sync_copy(src16.at[s], idx_s)
    pltpu.sync_copy(dst16.at[s], idx_d)

    def body(k, _):
        @pl.when(c == 0)
        def _():
            pltpu.sync_copy(hsA.at[idx_s.at[k]], rows_v)

        @pl.when(c == 1)
        def _():
            pltpu.sync_copy(hsB.at[idx_s.at[k]], rows_v)

        pltpu.sync_copy(rows_v, acc_sh.at[idx_d.at[k]], add=True)
        return 0

    lax.fori_loop(0, NCH1, body, 0)
    plsc.subcore_barrier()
    sl = pl.ds(s * ROWS_PER_SUB, ROWS_PER_SUB)
    pltpu.sync_copy(acc_sh.at[sl], out_hbm.at[c, sl])


@functools.partial(
    pl.kernel,
    out_type=jax.ShapeDtypeStruct((NC, NPAD, 64), jnp.float32),
    mesh=_MESH,
    compiler_params=pltpu.CompilerParams(use_tc_tiling_on_sc=False),
    scratch_types=[
        pltpu.VMEM((NCH1, CHUNK), jnp.int32),
        pltpu.VMEM((NCH1, CHUNK), jnp.int32),
        pltpu.VMEM((CHUNK, 64), jnp.float32),
        pltpu.VMEM((ZR, 64), jnp.float32),
        pltpu.VMEM_SHARED((NPAD, 64), jnp.float32),
    ],
    name="gcn_prop1",
)
def _prop1(hsA, hsB, src16, dst16, out_hbm, idx_s, idx_d, rows_v, zbuf_v,
           acc_sh):
    _prop1_body(hsA, hsB, src16, dst16, out_hbm, idx_s, idx_d, rows_v,
                zbuf_v, acc_sh)


def _prop_body(hs_hbm, src3_hbm, dst3_hbm, out_hbm, idx_s, idx_d, rows_v,
               zbuf_v, acc_sh, *, d):
    c = lax.axis_index("c")
    s = lax.axis_index("s")
    wid = s * NC + c
    _zero_acc(acc_sh, zbuf_v, s, d)
    plsc.subcore_barrier()

    pltpu.sync_copy(src3_hbm.at[wid], idx_s)
    pltpu.sync_copy(dst3_hbm.at[wid], idx_d)

    def body(k, _):
        pltpu.sync_copy(hs_hbm.at[idx_s.at[k]], rows_v)
        pltpu.sync_copy(rows_v, acc_sh.at[idx_d.at[k]], add=True)
        return 0

    lax.fori_loop(0, NCH, body, 0)
    plsc.subcore_barrier()
    sl = pl.ds(s * ROWS_PER_SUB, ROWS_PER_SUB)
    pltpu.sync_copy(acc_sh.at[sl], out_hbm.at[c, sl])


def _make_prop(d):
    @functools.partial(
        pl.kernel,
        out_type=jax.ShapeDtypeStruct((NC, NPAD, d), jnp.float32),
        mesh=_MESH,
        compiler_params=pltpu.CompilerParams(use_tc_tiling_on_sc=False),
        scratch_types=[
            pltpu.VMEM((NCH, CHUNK), jnp.int32),
            pltpu.VMEM((NCH, CHUNK), jnp.int32),
            pltpu.VMEM((CHUNK, d), jnp.float32),
            pltpu.VMEM((ZR, d), jnp.float32),
            pltpu.VMEM_SHARED((NPAD, d), jnp.float32),
        ],
        name=f"gcn_prop_{d}",
    )
    def prop(hs_hbm, src3_hbm, dst3_hbm, out_hbm, idx_s, idx_d, rows_v,
             zbuf_v, acc_sh):
        _prop_body(hs_hbm, src3_hbm, dst3_hbm, out_hbm, idx_s, idx_d,
                   rows_v, zbuf_v, acc_sh, d=d)

    return prop


_prop64 = _make_prop(64)
_prop16 = _make_prop(16)

BLK = 1024
GRID = NPAD // BLK


def _m1_body(x_ref, w_ref, d0_ref, d1_ref, hsA_ref, hsB_ref, dinv_ref):
    z = jnp.dot(x_ref[...], w_ref[...], preferred_element_type=jnp.float32)
    dinv = lax.rsqrt(d0_ref[...] + d1_ref[...] + 1.0)
    hs = z * dinv
    hsA_ref[...] = hs[:, :64]
    hsB_ref[...] = hs[:, 64:]
    dinv_ref[...] = dinv


def _m1(x, w1, d0, d1):
    din, dout = w1.shape
    return pl.pallas_call(
        _m1_body,
        grid=(GRID,),
        in_specs=[
            pl.BlockSpec((BLK, din), lambda i: (i, 0)),
            pl.BlockSpec((din, dout), lambda i: (0, 0)),
            pl.BlockSpec((BLK, 1), lambda i: (i, 0)),
            pl.BlockSpec((BLK, 1), lambda i: (i, 0)),
        ],
        out_specs=[
            pl.BlockSpec((BLK, 64), lambda i: (i, 0)),
            pl.BlockSpec((BLK, 64), lambda i: (i, 0)),
            pl.BlockSpec((BLK, 1), lambda i: (i, 0)),
        ],
        out_shape=[
            jax.ShapeDtypeStruct((NPAD, 64), jnp.float32),
            jax.ShapeDtypeStruct((NPAD, 64), jnp.float32),
            jax.ShapeDtypeStruct((NPAD, 1), jnp.float32),
        ],
    )(x, w1, d0, d1)


def _m2_body(a0_ref, a1_ref, hsA_ref, hsB_ref, dinv_ref, b_ref, w_ref,
             out_ref):
    dinv = dinv_ref[...]
    preA = dinv * (a0_ref[...] + hsA_ref[...]) + b_ref[:, :64]
    preB = dinv * (a1_ref[...] + hsB_ref[...]) + b_ref[:, 64:]
    h = jnp.concatenate([jnp.maximum(preA, 0.0), jnp.maximum(preB, 0.0)],
                        axis=1)
    z = jnp.dot(h, w_ref[...], preferred_element_type=jnp.float32)
    out_ref[...] = z * dinv


def _m2(a0, a1, hsA, hsB, dinv, b, w):
    din, dout = w.shape
    return pl.pallas_call(
        _m2_body,
        grid=(GRID,),
        in_specs=[
            pl.BlockSpec((BLK, 64), lambda i: (i, 0)),
            pl.BlockSpec((BLK, 64), lambda i: (i, 0)),
            pl.BlockSpec((BLK, 64), lambda i: (i, 0)),
            pl.BlockSpec((BLK, 64), lambda i: (i, 0)),
            pl.BlockSpec((BLK, 1), lambda i: (i, 0)),
            pl.BlockSpec((1, din), lambda i: (0, 0)),
            pl.BlockSpec((din, dout), lambda i: (0, 0)),
        ],
        out_specs=pl.BlockSpec((BLK, dout), lambda i: (i, 0)),
        out_shape=jax.ShapeDtypeStruct((NPAD, dout), jnp.float32),
    )(a0, a1, hsA, hsB, dinv, b, w)


def _m3_body(a0_ref, a1_ref, hs_ref, dinv_ref, b_ref, w_ref, out_ref):
    dinv = dinv_ref[...]
    pre = dinv * (a0_ref[...] + a1_ref[...] + hs_ref[...]) + b_ref[...]
    h = jnp.maximum(pre, 0.0)
    z = jnp.dot(h, w_ref[...], preferred_element_type=jnp.float32)
    out_ref[...] = z * dinv


def _m3(a0, a1, hs, dinv, b, w):
    din, dout = w.shape
    return pl.pallas_call(
        _m3_body,
        grid=(GRID,),
        in_specs=[
            pl.BlockSpec((BLK, din), lambda i: (i, 0)),
            pl.BlockSpec((BLK, din), lambda i: (i, 0)),
            pl.BlockSpec((BLK, din), lambda i: (i, 0)),
            pl.BlockSpec((BLK, 1), lambda i: (i, 0)),
            pl.BlockSpec((1, din), lambda i: (0, 0)),
            pl.BlockSpec((din, dout), lambda i: (0, 0)),
        ],
        out_specs=pl.BlockSpec((BLK, dout), lambda i: (i, 0)),
        out_shape=jax.ShapeDtypeStruct((NPAD, dout), jnp.float32),
    )(a0, a1, hs, dinv, b, w)


def _m4_body(a0_ref, a1_ref, hs_ref, dinv_ref, b_ref, out_ref):
    pre = (dinv_ref[...] * (a0_ref[...] + a1_ref[...] + hs_ref[...])
           + b_ref[...])
    m = jnp.max(pre, axis=1, keepdims=True)
    l = pre - m
    out_ref[...] = l - jnp.log(jnp.sum(jnp.exp(l), axis=1, keepdims=True))


def _m4(a0, a1, hs, dinv, b):
    d = hs.shape[1]
    return pl.pallas_call(
        _m4_body,
        grid=(GRID,),
        in_specs=[
            pl.BlockSpec((BLK, d), lambda i: (i, 0)),
            pl.BlockSpec((BLK, d), lambda i: (i, 0)),
            pl.BlockSpec((BLK, d), lambda i: (i, 0)),
            pl.BlockSpec((BLK, 1), lambda i: (i, 0)),
            pl.BlockSpec((1, d), lambda i: (0, 0)),
        ],
        out_specs=pl.BlockSpec((BLK, d), lambda i: (i, 0)),
        out_shape=jax.ShapeDtypeStruct((NPAD, d), jnp.float32),
    )(a0, a1, hs, dinv, b)


@jax.jit
def kernel(x, edge_index, W1, b1, W2, b2, W4, b4):
    src = edge_index[0]
    dst = edge_index[1]
    src3 = src.reshape(NW, NCH, CHUNK)
    dst3 = dst.reshape(NW, NCH, CHUNK)
    src16 = src.reshape(NS, NCH1, CHUNK)
    dst16 = dst.reshape(NS, NCH1, CHUNK)
    x_pad = jnp.pad(x, ((0, NPAD - N_NODES), (0, 0)))

    deg_p = _deg(dst3)
    d0 = deg_p[0].reshape(NPAD, 1)
    d1 = deg_p[1].reshape(NPAD, 1)

    hsA, hsB, dinv = _m1(x_pad, W1, d0, d1)
    agg1 = _prop1(hsA, hsB, src16, dst16)
    hs2 = _m2(agg1[0], agg1[1], hsA, hsB, dinv, b1.reshape(1, -1), W2)
    agg2 = _prop64(hs2, src3, dst3)
    hs3 = _m3(agg2[0], agg2[1], hs2, dinv, b2.reshape(1, -1), W4)
    agg3 = _prop16(hs3, src3, dst3)
    out = _m4(agg3[0], agg3[1], hs3, dinv, b4.reshape(1, -1))
    return out[:N_NODES]


# speedup vs baseline: 33.2263x; 1.1796x over previous
import functools

import jax
import jax.numpy as jnp
from jax import lax
from jax.experimental import pallas as pl
from jax.experimental.pallas import tpu as pltpu
from jax.experimental.pallas import tpu_sc as plsc

N_NODES = 10000
NPAD = 10240
N_EDGES = 320000
NC = 2
NS = 16
NW = NC * NS
CHUNK = 400
PER_W = N_EDGES // NW
NCH = PER_W // CHUNK
PER_S = N_EDGES // NS
NCH1 = PER_S // CHUNK
ROWS_PER_SUB = NPAD // NS
ZR = 64

_MESH = plsc.VectorSubcoreMesh(
    core_axis_name="c", subcore_axis_name="s", num_cores=NC, num_subcores=NS
)


def _fill_zeros(ref, nrows, d):
    per_row = d // 16

    def body(i, _):
        r = i // per_row
        col = (i % per_row) * 16
        ref[r, pl.ds(col, 16)] = jnp.zeros((16,), jnp.float32)
        return 0

    lax.fori_loop(0, nrows * per_row, body, 0)


def _zero_acc(acc_sh, zbuf_v, s, d):
    _fill_zeros(zbuf_v, ZR, d)

    def zcopy(j, _):
        pltpu.sync_copy(zbuf_v, acc_sh.at[pl.ds(s * ROWS_PER_SUB + j * ZR, ZR)])
        return 0

    lax.fori_loop(0, ROWS_PER_SUB // ZR, zcopy, 0)


def _deg_body(dst3_hbm, out_hbm, idx_v, ones_v, zbuf_v, deg_sh):
    c = lax.axis_index("c")
    s = lax.axis_index("s")
    wid = s * NC + c

    def fill_ones(i, _):
        ones_v[pl.ds(i * 16, 16)] = jnp.ones((16,), jnp.float32)
        return 0

    lax.fori_loop(0, CHUNK // 16, fill_ones, 0)

    def fill_z(i, _):
        zbuf_v[pl.ds(i * 16, 16)] = jnp.zeros((16,), jnp.float32)
        return 0

    lax.fori_loop(0, ROWS_PER_SUB // 16, fill_z, 0)
    pltpu.sync_copy(zbuf_v, deg_sh.at[pl.ds(s * ROWS_PER_SUB, ROWS_PER_SUB)])
    plsc.subcore_barrier()

    pltpu.sync_copy(dst3_hbm.at[wid], idx_v)

    def body(k, _):
        pltpu.sync_copy(ones_v, deg_sh.at[idx_v.at[k]], add=True)
        return 0

    lax.fori_loop(0, NCH, body, 0)
    plsc.subcore_barrier()
    sl = pl.ds(s * ROWS_PER_SUB, ROWS_PER_SUB)
    pltpu.sync_copy(deg_sh.at[sl], out_hbm.at[c, sl])


@functools.partial(
    pl.kernel,
    out_type=jax.ShapeDtypeStruct((NC, NPAD), jnp.float32),
    mesh=_MESH,
    compiler_params=pltpu.CompilerParams(use_tc_tiling_on_sc=False),
    scratch_types=[
        pltpu.VMEM((NCH, CHUNK), jnp.int32),
        pltpu.VMEM((CHUNK,), jnp.float32),
        pltpu.VMEM((ROWS_PER_SUB,), jnp.float32),
        pltpu.VMEM_SHARED((NPAD,), jnp.float32),
    ],
    name="gcn_deg",
)
def _deg(dst3_hbm, out_hbm, idx_v, ones_v, zbuf_v, deg_sh):
    _deg_body(dst3_hbm, out_hbm, idx_v, ones_v, zbuf_v, deg_sh)


def _edge_loop(nch, load_idx, wait_idx, start_gather, wait_gather, scatter):
    load_idx(0, 0)
    wait_idx(0)
    start_gather(0)
    load_idx(1, 1)
    n_pairs = (nch - 2) // 2

    def step(k, b):
        wait_gather(b)
        wait_idx(1 - b)
        start_gather(1 - b)
        scatter(b)
        load_idx(k + 2, b)

    def pair(i, _):
        step(i * 2, 0)
        step(i * 2 + 1, 1)
        return 0

    lax.fori_loop(0, n_pairs, pair, 0)
    for k in range(2 * n_pairs, nch):
        b = k % 2
        wait_gather(b)
        if k + 1 < nch:
            wait_idx(1 - b)
            start_gather(1 - b)
        scatter(b)
        if k + 2 < nch:
            load_idx(k + 2, b)


def _prop_common(gather_from, hs_like, src_slab, dst_slab, out_hbm, c, s,
                 nch, idx_sb, idx_db, rows_a, rows_b, zbuf_v, acc_sh,
                 gsem0, gsem1, isem0, isem1, d):
    _zero_acc(acc_sh, zbuf_v, s, d)
    plsc.subcore_barrier()

    rows = (rows_a, rows_b)
    gsems = (gsem0, gsem1)
    isems = (isem0, isem1)

    def load_idx(k, b):
        pltpu.async_copy(src_slab.at[k], idx_sb.at[b], isems[b])
        pltpu.async_copy(dst_slab.at[k], idx_db.at[b], isems[b])

    def wait_idx(b):
        pltpu.make_async_copy(src_slab.at[0], idx_sb.at[b], isems[b]).wait()
        pltpu.make_async_copy(dst_slab.at[0], idx_db.at[b], isems[b]).wait()

    def start_gather(b):
        gather_from(idx_sb.at[b], rows[b], gsems[b])

    def wait_gather(b):
        pltpu.make_async_copy(hs_like.at[idx_sb.at[b]], rows[b],
                              gsems[b]).wait()

    def scatter(b):
        pltpu.sync_copy(rows[b], acc_sh.at[idx_db.at[b]], add=True)

    _edge_loop(nch, load_idx, wait_idx, start_gather, wait_gather, scatter)
    plsc.subcore_barrier()
    sl = pl.ds(s * ROWS_PER_SUB, ROWS_PER_SUB)
    pltpu.sync_copy(acc_sh.at[sl], out_hbm.at[c, sl])


def _prop_scratch(d):
    return [
        pltpu.VMEM((2, CHUNK), jnp.int32),
        pltpu.VMEM((2, CHUNK), jnp.int32),
        pltpu.VMEM((CHUNK, d), jnp.float32),
        pltpu.VMEM((CHUNK, d), jnp.float32),
        pltpu.VMEM((ZR, d), jnp.float32),
        pltpu.VMEM_SHARED((NPAD, d), jnp.float32),
        pltpu.SemaphoreType.DMA,
        pltpu.SemaphoreType.DMA,
        pltpu.SemaphoreType.DMA,
        pltpu.SemaphoreType.DMA,
    ]


@functools.partial(
    pl.kernel,
    out_type=jax.ShapeDtypeStruct((NC, NPAD, 64), jnp.float32),
    mesh=_MESH,
    compiler_params=pltpu.CompilerParams(use_tc_tiling_on_sc=False),
    scratch_types=_prop_scratch(64),
    name="gcn_prop1",
)
def _prop1(hsA, hsB, src16, dst16, out_hbm, idx_sb, idx_db, rows_a, rows_b,
           zbuf_v, acc_sh, gsem0, gsem1, isem0, isem1):
    c = lax.axis_index("c")
    s = lax.axis_index("s")

    def gather_from(idx_ref, dst_ref, sem):
        @pl.when(c == 0)
        def _():
            pltpu.async_copy(hsA.at[idx_ref], dst_ref, sem)

        @pl.when(c == 1)
        def _():
            pltpu.async_copy(hsB.at[idx_ref], dst_ref, sem)

    _prop_common(gather_from, hsA, src16.at[s], dst16.at[s], out_hbm, c, s,
                 NCH1, idx_sb, idx_db, rows_a, rows_b, zbuf_v, acc_sh,
                 gsem0, gsem1, isem0, isem1, 64)


def _make_prop(d):
    @functools.partial(
        pl.kernel,
        out_type=jax.ShapeDtypeStruct((NC, NPAD, d), jnp.float32),
        mesh=_MESH,
        compiler_params=pltpu.CompilerParams(use_tc_tiling_on_sc=False),
        scratch_types=_prop_scratch(d),
        name=f"gcn_prop_{d}",
    )
    def prop(hs_hbm, src3_hbm, dst3_hbm, out_hbm, idx_sb, idx_db, rows_a,
             rows_b, zbuf_v, acc_sh, gsem0, gsem1, isem0, isem1):
        c = lax.axis_index("c")
        s = lax.axis_index("s")
        wid = s * NC + c

        def gather_from(idx_ref, dst_ref, sem):
            pltpu.async_copy(hs_hbm.at[idx_ref], dst_ref, sem)

        _prop_common(gather_from, hs_hbm, src3_hbm.at[wid], dst3_hbm.at[wid],
                     out_hbm, c, s, NCH, idx_sb, idx_db, rows_a, rows_b,
                     zbuf_v, acc_sh, gsem0, gsem1, isem0, isem1, d)

    return prop


_prop64 = _make_prop(64)
_prop16 = _make_prop(16)


BLK = 1024
GRID = NPAD // BLK


def _m1_body(x_ref, w_ref, d0_ref, d1_ref, hsA_ref, hsB_ref, dinv_ref):
    z = jnp.dot(x_ref[...], w_ref[...], preferred_element_type=jnp.float32)
    dinv = lax.rsqrt(d0_ref[...] + d1_ref[...] + 1.0)
    hs = z * dinv
    hsA_ref[...] = hs[:, :64]
    hsB_ref[...] = hs[:, 64:]
    dinv_ref[...] = dinv


def _m1(x, w1, d0, d1):
    din, dout = w1.shape
    return pl.pallas_call(
        _m1_body,
        grid=(GRID,),
        in_specs=[
            pl.BlockSpec((BLK, din), lambda i: (i, 0)),
            pl.BlockSpec((din, dout), lambda i: (0, 0)),
            pl.BlockSpec((BLK, 1), lambda i: (i, 0)),
            pl.BlockSpec((BLK, 1), lambda i: (i, 0)),
        ],
        out_specs=[
            pl.BlockSpec((BLK, 64), lambda i: (i, 0)),
            pl.BlockSpec((BLK, 64), lambda i: (i, 0)),
            pl.BlockSpec((BLK, 1), lambda i: (i, 0)),
        ],
        out_shape=[
            jax.ShapeDtypeStruct((NPAD, 64), jnp.float32),
            jax.ShapeDtypeStruct((NPAD, 64), jnp.float32),
            jax.ShapeDtypeStruct((NPAD, 1), jnp.float32),
        ],
    )(x, w1, d0, d1)


def _m2_body(a0_ref, a1_ref, hsA_ref, hsB_ref, dinv_ref, b_ref, w_ref,
             out_ref):
    dinv = dinv_ref[...]
    preA = dinv * (a0_ref[...] + hsA_ref[...]) + b_ref[:, :64]
    preB = dinv * (a1_ref[...] + hsB_ref[...]) + b_ref[:, 64:]
    h = jnp.concatenate([jnp.maximum(preA, 0.0), jnp.maximum(preB, 0.0)],
                        axis=1)
    z = jnp.dot(h, w_ref[...], preferred_element_type=jnp.float32)
    out_ref[...] = z * dinv


def _m2(a0, a1, hsA, hsB, dinv, b, w):
    din, dout = w.shape
    return pl.pallas_call(
        _m2_body,
        grid=(GRID,),
        in_specs=[
            pl.BlockSpec((BLK, 64), lambda i: (i, 0)),
            pl.BlockSpec((BLK, 64), lambda i: (i, 0)),
            pl.BlockSpec((BLK, 64), lambda i: (i, 0)),
            pl.BlockSpec((BLK, 64), lambda i: (i, 0)),
            pl.BlockSpec((BLK, 1), lambda i: (i, 0)),
            pl.BlockSpec((1, din), lambda i: (0, 0)),
            pl.BlockSpec((din, dout), lambda i: (0, 0)),
        ],
        out_specs=pl.BlockSpec((BLK, dout), lambda i: (i, 0)),
        out_shape=jax.ShapeDtypeStruct((NPAD, dout), jnp.float32),
    )(a0, a1, hsA, hsB, dinv, b, w)


def _m3_body(a0_ref, a1_ref, hs_ref, dinv_ref, b_ref, w_ref, out_ref):
    dinv = dinv_ref[...]
    pre = dinv * (a0_ref[...] + a1_ref[...] + hs_ref[...]) + b_ref[...]
    h = jnp.maximum(pre, 0.0)
    z = jnp.dot(h, w_ref[...], preferred_element_type=jnp.float32)
    out_ref[...] = z * dinv


def _m3(a0, a1, hs, dinv, b, w):
    din, dout = w.shape
    return pl.pallas_call(
        _m3_body,
        grid=(GRID,),
        in_specs=[
            pl.BlockSpec((BLK, din), lambda i: (i, 0)),
            pl.BlockSpec((BLK, din), lambda i: (i, 0)),
            pl.BlockSpec((BLK, din), lambda i: (i, 0)),
            pl.BlockSpec((BLK, 1), lambda i: (i, 0)),
            pl.BlockSpec((1, din), lambda i: (0, 0)),
            pl.BlockSpec((din, dout), lambda i: (0, 0)),
        ],
        out_specs=pl.BlockSpec((BLK, dout), lambda i: (i, 0)),
        out_shape=jax.ShapeDtypeStruct((NPAD, dout), jnp.float32),
    )(a0, a1, hs, dinv, b, w)


def _m4_body(a0_ref, a1_ref, hs_ref, dinv_ref, b_ref, out_ref):
    pre = (dinv_ref[...] * (a0_ref[...] + a1_ref[...] + hs_ref[...])
           + b_ref[...])
    m = jnp.max(pre, axis=1, keepdims=True)
    l = pre - m
    out_ref[...] = l - jnp.log(jnp.sum(jnp.exp(l), axis=1, keepdims=True))


def _m4(a0, a1, hs, dinv, b):
    d = hs.shape[1]
    return pl.pallas_call(
        _m4_body,
        grid=(GRID,),
        in_specs=[
            pl.BlockSpec((BLK, d), lambda i: (i, 0)),
            pl.BlockSpec((BLK, d), lambda i: (i, 0)),
            pl.BlockSpec((BLK, d), lambda i: (i, 0)),
            pl.BlockSpec((BLK, 1), lambda i: (i, 0)),
            pl.BlockSpec((1, d), lambda i: (0, 0)),
        ],
        out_specs=pl.BlockSpec((BLK, d), lambda i: (i, 0)),
        out_shape=jax.ShapeDtypeStruct((NPAD, d), jnp.float32),
    )(a0, a1, hs, dinv, b)


@jax.jit
def kernel(x, edge_index, W1, b1, W2, b2, W4, b4):
    src = edge_index[0]
    dst = edge_index[1]
    src3 = src.reshape(NW, NCH, CHUNK)
    dst3 = dst.reshape(NW, NCH, CHUNK)
    src16 = src.reshape(NS, NCH1, CHUNK)
    dst16 = dst.reshape(NS, NCH1, CHUNK)
    x_pad = jnp.pad(x, ((0, NPAD - N_NODES), (0, 0)))

    deg_p = _deg(dst3)
    d0 = deg_p[0].reshape(NPAD, 1)
    d1 = deg_p[1].reshape(NPAD, 1)

    hsA, hsB, dinv = _m1(x_pad, W1, d0, d1)
    agg1 = _prop1(hsA, hsB, src16, dst16)
    hs2 = _m2(agg1[0], agg1[1], hsA, hsB, dinv, b1.reshape(1, -1), W2)
    agg2 = _prop64(hs2, src3, dst3)
    hs3 = _m3(agg2[0], agg2[1], hs2, dinv, b2.reshape(1, -1), W4)
    agg3 = _prop16(hs3, src3, dst3)
    out = _m4(agg3[0], agg3[1], hs3, dinv, b4.reshape(1, -1))
    return out[:N_NODES]
